# baseline (speedup 1.0000x reference)
import functools

import jax
import jax.numpy as jnp
from jax import lax
from jax.experimental import pallas as pl
from jax.experimental.pallas import tpu as pltpu
from jax.experimental.pallas import tpu_sc as plsc

N = 10000
E = 320000
D = 128
B = 256
DIM = 32
C = 10

NC = 2
NS = 16
NW = NC * NS

N_PAD = 10240
NPT = N_PAD // NW
EPT = E // NW
K = 256
NCH = (EPT + K - 1) // K
TRASH = N + 8
BP = 384
BCH = (BP + K - 1) // K
NBUF = 6
PD = 3

f32 = jnp.float32


def _sds(shape, dtype=f32):
    return jax.ShapeDtypeStruct(shape, dtype)


_MESH = plsc.VectorSubcoreMesh(
    core_axis_name="c", subcore_axis_name="s", num_cores=NC, num_subcores=NS
)


def _degcnt_body(dstp, batc, zn, deg_out, cnt_out, didx, bidx, ones_v, deg_sh, cnt_sh):
    c = lax.axis_index("c")
    s = lax.axis_index("s")
    wid = c * NS + s

    for i in range(K // 16):
        ones_v[pl.ds(i * 16, 16)] = jnp.ones((16,), f32)

    pltpu.sync_copy(zn, deg_sh.at[pl.ds(s * (N_PAD // NS), N_PAD // NS)])

    @pl.when(s == 0)
    def _():
        pltpu.sync_copy(zn.at[pl.ds(0, BP)], cnt_sh)

    pltpu.sync_copy(dstp.at[wid], didx)
    pltpu.sync_copy(batc.at[wid], bidx)
    plsc.subcore_barrier()

    def chunk(j, carry):
        pltpu.sync_copy(ones_v, deg_sh.at[didx.at[j]], add=True)
        return carry

    lax.fori_loop(0, NCH, chunk, 0)

    def bchunk(j, carry):
        pltpu.sync_copy(ones_v, cnt_sh.at[bidx.at[j]], add=True)
        return carry

    lax.fori_loop(0, BCH, bchunk, 0)

    plsc.subcore_barrier()
    pltpu.sync_copy(
        deg_sh.at[pl.ds(s * (N_PAD // NS), N_PAD // NS)],
        deg_out.at[c, pl.ds(s * (N_PAD // NS), N_PAD // NS)],
    )

    @pl.when(s == 0)
    def _():
        pltpu.sync_copy(cnt_sh, cnt_out.at[c])


_SC_PARAMS = pltpu.CompilerParams(
    use_tc_tiling_on_sc=False, needs_layout_passes=False
)

_sc_degcnt = pl.kernel(
    _degcnt_body,
    out_type=(_sds((NC, N_PAD)), _sds((NC, BP))),
    mesh=_MESH,
    compiler_params=_SC_PARAMS,
    scratch_types=(
        pltpu.VMEM((NCH, K), jnp.int32),
        pltpu.VMEM((BCH, K), jnp.int32),
        pltpu.VMEM((K,), f32),
        pltpu.VMEM_SHARED((N_PAD,), f32),
        pltpu.VMEM_SHARED((BP,), f32),
    ),
)


def _agg_pool_body(do_agg, do_pool, *refs):
    refs = list(refs)
    if do_agg:
        hn, srcp, dstp, z640 = refs[:4]
        del refs[:4]
    if do_pool:
        hprev_f, batp, zf, negf = refs[:4]
        del refs[:4]
    if do_agg:
        aggp = refs.pop(0)
    if do_pool:
        psump, pmaxp = refs[:2]
        del refs[:2]
    if do_agg:
        sidx, didx, rows, gsem, ssem = refs[:5]
        del refs[:5]
    if do_pool:
        hbuf, bbuf, psum, pmax = refs[:4]
        del refs[:4]
    if do_agg:
        agg_sh = refs.pop(0)

    c = lax.axis_index("c")
    s = lax.axis_index("s")
    wid = c * NS + s

    if do_agg:
        pltpu.sync_copy(z640, agg_sh.at[pl.ds(s * (N_PAD // NS), N_PAD // NS)])
        pltpu.sync_copy(srcp.at[wid], sidx)
        pltpu.sync_copy(dstp.at[wid], didx)
    if do_pool:
        pltpu.sync_copy(hprev_f.at[pl.ds(wid * NPT * DIM, NPT * DIM)], hbuf)
        pltpu.sync_copy(batp.at[wid], bbuf)
        pltpu.sync_copy(zf, psum)
        pltpu.sync_copy(negf, pmax)
    if do_agg:
        plsc.subcore_barrier()

        for b in range(PD):
            pltpu.async_copy(hn.at[sidx.at[b]], rows.at[b], gsem.at[b])

        def chunk(j, carry):
            slot = j % NBUF
            nxt = j + PD

            @pl.when(jnp.logical_and(nxt < NCH, j >= PD))
            def _():
                pltpu.make_async_copy(
                    rows.at[nxt % NBUF], agg_sh.at[didx.at[j]], ssem.at[nxt % NBUF]
                ).wait()

            @pl.when(nxt < NCH)
            def _():
                pltpu.async_copy(
                    hn.at[sidx.at[nxt]], rows.at[nxt % NBUF], gsem.at[nxt % NBUF]
                )

            pltpu.make_async_copy(
                hn.at[sidx.at[j]], rows.at[slot], gsem.at[slot]
            ).wait()
            pltpu.async_copy(
                rows.at[slot], agg_sh.at[didx.at[j]], ssem.at[slot], add=True
            )
            return carry

        lax.fori_loop(0, NCH, chunk, 0)
        for t in range(max(0, NCH - 2 * PD), NCH):
            pltpu.make_async_copy(
                rows.at[t % NBUF], agg_sh.at[didx.at[t]], ssem.at[t % NBUF]
            ).wait()

    if do_pool:
        lanes = lax.iota(jnp.int32, 16)

        def pnode(i, carry):
            bi = plsc.load_gather(bbuf, [jnp.full((16,), i, jnp.int32)])
            r0 = plsc.load_gather(hbuf, [i * DIM + lanes])
            r1 = plsc.load_gather(hbuf, [i * DIM + 16 + lanes])
            i0 = bi * DIM + lanes
            i1 = i0 + 16
            plsc.addupdate_scatter(psum, [i0], r0)
            plsc.addupdate_scatter(psum, [i1], r1)
            plsc.store_scatter(
                pmax, [i0], jnp.maximum(plsc.load_gather(pmax, [i0]), r0)
            )
            plsc.store_scatter(
                pmax, [i1], jnp.maximum(plsc.load_gather(pmax, [i1]), r1)
            )
            return carry

        lax.fori_loop(0, NPT, pnode, 0)

    if do_agg:
        plsc.subcore_barrier()
        pltpu.sync_copy(
            agg_sh.at[pl.ds(s * (N_PAD // NS), N_PAD // NS)],
            aggp.at[c, pl.ds(s * (N_PAD // NS), N_PAD // NS)],
        )
    if do_pool:
        pltpu.sync_copy(psum, psump.at[wid])
        pltpu.sync_copy(pmax, pmaxp.at[wid])


def _make_agg_pool(do_agg, do_pool):
    out_type = []
    scratch = []
    if do_agg:
        out_type.append(_sds((NC, N_PAD, DIM)))
        scratch += [
            pltpu.VMEM((NCH, K), jnp.int32),
            pltpu.VMEM((NCH, K), jnp.int32),
            pltpu.VMEM((NBUF, K, DIM), f32),
            pltpu.SemaphoreType.DMA((NBUF,)),
            pltpu.SemaphoreType.DMA((NBUF,)),
        ]
    if do_pool:
        out_type += [_sds((NW, BP * DIM)), _sds((NW, BP * DIM))]
        scratch += [
            pltpu.VMEM((NPT * DIM,), f32),
            pltpu.VMEM((BP,), jnp.int32),
            pltpu.VMEM((BP * DIM,), f32),
            pltpu.VMEM((BP * DIM,), f32),
        ]
    if do_agg:
        scratch.append(pltpu.VMEM_SHARED((N_PAD, DIM), f32))
    return pl.kernel(
        functools.partial(_agg_pool_body, do_agg, do_pool),
        out_type=tuple(out_type),
        mesh=_MESH,
        compiler_params=_SC_PARAMS,
        scratch_types=tuple(scratch),
    )


_sc_agg = _make_agg_pool(True, False)
_sc_agg_pool = _make_agg_pool(True, True)
_sc_pool = _make_agg_pool(False, True)


_RB = 1024
_GRID = N_PAD // _RB


def _t1_body(degp, x, w1, dinv, hn):
    deg = degp[0] + degp[1] + 1.0
    di = lax.rsqrt(deg)
    dinv[...] = di
    hn[...] = jnp.dot(x[...], w1[...], preferred_element_type=f32) * di


def _t1(degp, x_pad, W1):
    return pl.pallas_call(
        _t1_body,
        grid=(_GRID,),
        in_specs=[
            pl.BlockSpec((NC, _RB, 1), lambda i: (0, i, 0)),
            pl.BlockSpec((_RB, D), lambda i: (i, 0)),
            pl.BlockSpec((D, DIM), lambda i: (0, 0)),
        ],
        out_specs=[
            pl.BlockSpec((_RB, 1), lambda i: (i, 0)),
            pl.BlockSpec((_RB, DIM), lambda i: (i, 0)),
        ],
        out_shape=[_sds((N_PAD, 1)), _sds((N_PAD, DIM))],
    )(degp, x_pad, W1)


def _tmid_body(aggp, hn, dinv, b, w, hout, hnextn):
    di = dinv[...]
    h = jnp.maximum(di * (aggp[0] + aggp[1] + hn[...]) + b[...], 0.0)
    hout[...] = h
    hnextn[...] = jnp.dot(h, w[...], preferred_element_type=f32) * di


def _tmid(aggp, hn, dinv, b, Wnext):
    return pl.pallas_call(
        _tmid_body,
        grid=(_GRID,),
        in_specs=[
            pl.BlockSpec((NC, _RB, DIM), lambda i: (0, i, 0)),
            pl.BlockSpec((_RB, DIM), lambda i: (i, 0)),
            pl.BlockSpec((_RB, 1), lambda i: (i, 0)),
            pl.BlockSpec((1, DIM), lambda i: (0, 0)),
            pl.BlockSpec((DIM, DIM), lambda i: (0, 0)),
        ],
        out_specs=[
            pl.BlockSpec((_RB, DIM), lambda i: (i, 0)),
            pl.BlockSpec((_RB, DIM), lambda i: (i, 0)),
        ],
        out_shape=[_sds((N_PAD, DIM)), _sds((N_PAD, DIM))],
    )(aggp, hn, dinv, b, Wnext)


def _tlast_body(aggp, hn, dinv, b, hout):
    h = jnp.maximum(dinv[...] * (aggp[0] + aggp[1] + hn[...]) + b[...], 0.0)
    hout[...] = h


def _tlast(aggp, hn, dinv, b):
    return pl.pallas_call(
        _tlast_body,
        grid=(_GRID,),
        in_specs=[
            pl.BlockSpec((NC, _RB, DIM), lambda i: (0, i, 0)),
            pl.BlockSpec((_RB, DIM), lambda i: (i, 0)),
            pl.BlockSpec((_RB, 1), lambda i: (i, 0)),
            pl.BlockSpec((1, DIM), lambda i: (0, 0)),
        ],
        out_specs=pl.BlockSpec((_RB, DIM), lambda i: (i, 0)),
        out_shape=_sds((N_PAD, DIM)),
    )(aggp, hn, dinv, b)


def _head_body(ps1, pm1, ps2, pm2, ps3, pm3, cntp, wl1, bl1, wl2, bl2, wl3, bl3, out):
    cnt = jnp.maximum(cntp[0] + cntp[1], 1.0)

    def pooled(ps, pm):
        sm = jnp.sum(ps[...], axis=0)
        mx = jnp.max(pm[...], axis=0)
        mx = jnp.where(jnp.isfinite(mx), mx, 0.0)
        xl = jnp.concatenate([mx, sm / cnt], axis=1)
        return jnp.maximum(xl, 0.0)

    g = pooled(ps1, pm1) + pooled(ps2, pm2) + pooled(ps3, pm3)
    g = jnp.maximum(jnp.dot(g, wl1[...], preferred_element_type=f32) + bl1[...], 0.0)
    g = jnp.maximum(jnp.dot(g, wl2[...], preferred_element_type=f32) + bl2[...], 0.0)
    lg = jnp.dot(g, wl3[...], preferred_element_type=f32) + bl3[...]
    m = jnp.max(lg, axis=-1, keepdims=True)
    lse = jnp.log(jnp.sum(jnp.exp(lg - m), axis=-1, keepdims=True))
    out[...] = (lg - m - lse)[:B]


def _head(ps1, pm1, ps2, pm2, ps3, pm3, cntp, Wl1, bl1, Wl2, bl2, Wl3, bl3):
    return pl.pallas_call(_head_body, out_shape=_sds((B, C)))(
        ps1, pm1, ps2, pm2, ps3, pm3, cntp, Wl1, bl1, Wl2, bl2, Wl3, bl3
    )


def kernel(x, edge_index, batch, W1, b1, W2, b2, W3, b3, Wl1, bl1, Wl2, bl2, Wl3, bl3):
    src = edge_index[0]
    dst = edge_index[1]
    pad_e = NCH * K - EPT
    srcp = jnp.pad(
        src.reshape(NW, EPT), ((0, 0), (0, pad_e)), constant_values=0
    ).reshape(NW, NCH, K)
    dstp = jnp.pad(
        dst.reshape(NW, EPT), ((0, 0), (0, pad_e)), constant_values=TRASH
    ).reshape(NW, NCH, K)
    batp = jnp.pad(
        jnp.pad(batch, (0, N_PAD - N), constant_values=B).reshape(NW, NPT),
        ((0, 0), (0, BP - NPT)),
        constant_values=B,
    )
    batc = jnp.pad(batch, (0, NW * BCH * K - N), constant_values=B).reshape(
        NW, BCH, K
    )
    x_pad = jnp.pad(x, ((0, N_PAD - N), (0, 0)))

    zn = jnp.zeros((N_PAD // NS,), f32)
    z640 = jnp.zeros((N_PAD // NS, DIM), f32)
    zf = jnp.zeros((BP * DIM,), f32)
    negf = jnp.full((BP * DIM,), -jnp.inf, f32)

    degp, cntp = _sc_degcnt(dstp, batc, zn)

    dinv, h1n = _t1(degp.reshape(NC, N_PAD, 1), x_pad, W1)
    (aggp1,) = _sc_agg(h1n, srcp, dstp, z640)
    h1, h2n = _tmid(aggp1, h1n, dinv, b1.reshape(1, DIM), W2)

    aggp2, ps1, pm1 = _sc_agg_pool(
        h2n, srcp, dstp, z640, h1.reshape(-1), batp, zf, negf
    )
    h2, h3n = _tmid(aggp2, h2n, dinv, b2.reshape(1, DIM), W3)

    aggp3, ps2, pm2 = _sc_agg_pool(
        h3n, srcp, dstp, z640, h2.reshape(-1), batp, zf, negf
    )
    h3 = _tlast(aggp3, h3n, dinv, b3.reshape(1, DIM))

    ps3, pm3 = _sc_pool(h3.reshape(-1), batp, zf, negf)

    def prs(p):
        return p.reshape(NW, BP, DIM)

    return _head(
        prs(ps1), prs(pm1), prs(ps2), prs(pm2), prs(ps3), prs(pm3),
        cntp.reshape(NC, BP, 1),
        Wl1, bl1.reshape(1, DIM), Wl2, bl2.reshape(1, DIM // 2),
        Wl3, bl3.reshape(1, C),
    )


# speedup vs baseline: 1.2893x; 1.2893x over previous
import functools

import jax
import jax.numpy as jnp
from jax import lax
from jax.experimental import pallas as pl
from jax.experimental.pallas import tpu as pltpu
from jax.experimental.pallas import tpu_sc as plsc

N = 10000
E = 320000
D = 128
B = 256
DIM = 32
C = 10

NC = 2
NS = 16
NW = NC * NS

N_PAD = 10240
NPT = N_PAD // NW
EPT = E // NW
K = 128
NCH = (EPT + K - 1) // K
TRASH = N + 8
BP = 384
BCH = (BP + K - 1) // K
NBUF = 12
PD = 6

f32 = jnp.float32


def _sds(shape, dtype=f32):
    return jax.ShapeDtypeStruct(shape, dtype)


_MESH = plsc.VectorSubcoreMesh(
    core_axis_name="c", subcore_axis_name="s", num_cores=NC, num_subcores=NS
)


def _degcnt_body(dstp, batc, zn, deg_out, cnt_out, didx, bidx, ones_v, deg_sh, cnt_sh):
    c = lax.axis_index("c")
    s = lax.axis_index("s")
    wid = c * NS + s

    for i in range(K // 16):
        ones_v[pl.ds(i * 16, 16)] = jnp.ones((16,), f32)

    pltpu.sync_copy(zn, deg_sh.at[pl.ds(s * (N_PAD // NS), N_PAD // NS)])

    @pl.when(s == 0)
    def _():
        pltpu.sync_copy(zn.at[pl.ds(0, BP)], cnt_sh)

    pltpu.sync_copy(dstp.at[wid], didx)
    pltpu.sync_copy(batc.at[wid], bidx)
    plsc.subcore_barrier()

    def chunk(j, carry):
        pltpu.sync_copy(ones_v, deg_sh.at[didx.at[j]], add=True)
        return carry

    lax.fori_loop(0, NCH, chunk, 0)

    def bchunk(j, carry):
        pltpu.sync_copy(ones_v, cnt_sh.at[bidx.at[j]], add=True)
        return carry

    lax.fori_loop(0, BCH, bchunk, 0)

    plsc.subcore_barrier()
    pltpu.sync_copy(
        deg_sh.at[pl.ds(s * (N_PAD // NS), N_PAD // NS)],
        deg_out.at[c, pl.ds(s * (N_PAD // NS), N_PAD // NS)],
    )

    @pl.when(s == 0)
    def _():
        pltpu.sync_copy(cnt_sh, cnt_out.at[c])


_SC_PARAMS = pltpu.CompilerParams(
    use_tc_tiling_on_sc=False, needs_layout_passes=False
)

_sc_degcnt = pl.kernel(
    _degcnt_body,
    out_type=(_sds((NC, N_PAD)), _sds((NC, BP))),
    mesh=_MESH,
    compiler_params=_SC_PARAMS,
    scratch_types=(
        pltpu.VMEM((NCH, K), jnp.int32),
        pltpu.VMEM((BCH, K), jnp.int32),
        pltpu.VMEM((K,), f32),
        pltpu.VMEM_SHARED((N_PAD,), f32),
        pltpu.VMEM_SHARED((BP,), f32),
    ),
)


def _agg_pool_body(do_agg, do_pool, *refs):
    refs = list(refs)
    if do_agg:
        hn, srcp, dstp, z640 = refs[:4]
        del refs[:4]
    if do_pool:
        hprev_f, batp, zf, negf = refs[:4]
        del refs[:4]
    if do_agg:
        aggp = refs.pop(0)
    if do_pool:
        psump, pmaxp = refs[:2]
        del refs[:2]
    if do_agg:
        sidx, didx, rows, gsem, ssem = refs[:5]
        del refs[:5]
    if do_pool:
        hbuf, bbuf, psum, pmax = refs[:4]
        del refs[:4]
    if do_agg:
        agg_sh = refs.pop(0)

    c = lax.axis_index("c")
    s = lax.axis_index("s")
    wid = c * NS + s

    if do_agg:
        pltpu.sync_copy(z640, agg_sh.at[pl.ds(s * (N_PAD // NS), N_PAD // NS)])
        pltpu.sync_copy(srcp.at[wid], sidx)
        pltpu.sync_copy(dstp.at[wid], didx)
    if do_pool:
        pltpu.sync_copy(hprev_f.at[pl.ds(wid * NPT * DIM, NPT * DIM)], hbuf)
        pltpu.sync_copy(batp.at[wid], bbuf)
        pltpu.sync_copy(zf, psum)
        pltpu.sync_copy(negf, pmax)
    if do_agg:
        plsc.subcore_barrier()

        for b in range(PD):
            pltpu.async_copy(hn.at[sidx.at[b]], rows.at[b], gsem.at[b])

        def chunk(j, carry):
            slot = j % NBUF
            nxt = j + PD

            @pl.when(jnp.logical_and(nxt < NCH, j >= PD))
            def _():
                pltpu.make_async_copy(
                    rows.at[nxt % NBUF], agg_sh.at[didx.at[j]], ssem.at[nxt % NBUF]
                ).wait()

            @pl.when(nxt < NCH)
            def _():
                pltpu.async_copy(
                    hn.at[sidx.at[nxt]], rows.at[nxt % NBUF], gsem.at[nxt % NBUF]
                )

            pltpu.make_async_copy(
                hn.at[sidx.at[j]], rows.at[slot], gsem.at[slot]
            ).wait()
            pltpu.async_copy(
                rows.at[slot], agg_sh.at[didx.at[j]], ssem.at[slot], add=True
            )
            return carry

        lax.fori_loop(0, NCH, chunk, 0)
        for t in range(max(0, NCH - 2 * PD), NCH):
            pltpu.make_async_copy(
                rows.at[t % NBUF], agg_sh.at[didx.at[t]], ssem.at[t % NBUF]
            ).wait()

    if do_pool:
        lanes = lax.iota(jnp.int32, 16)

        def pnode(i, carry):
            bi = plsc.load_gather(bbuf, [jnp.full((16,), i, jnp.int32)])
            r0 = plsc.load_gather(hbuf, [i * DIM + lanes])
            r1 = plsc.load_gather(hbuf, [i * DIM + 16 + lanes])
            i0 = bi * DIM + lanes
            i1 = i0 + 16
            plsc.addupdate_scatter(psum, [i0], r0)
            plsc.addupdate_scatter(psum, [i1], r1)
            plsc.store_scatter(
                pmax, [i0], jnp.maximum(plsc.load_gather(pmax, [i0]), r0)
            )
            plsc.store_scatter(
                pmax, [i1], jnp.maximum(plsc.load_gather(pmax, [i1]), r1)
            )
            return carry

        lax.fori_loop(0, NPT, pnode, 0)

    if do_agg:
        plsc.subcore_barrier()
        pltpu.sync_copy(
            agg_sh.at[pl.ds(s * (N_PAD // NS), N_PAD // NS)],
            aggp.at[c, pl.ds(s * (N_PAD // NS), N_PAD // NS)],
        )
    if do_pool:
        pltpu.sync_copy(psum, psump.at[wid])
        pltpu.sync_copy(pmax, pmaxp.at[wid])


def _make_agg_pool(do_agg, do_pool):
    out_type = []
    scratch = []
    if do_agg:
        out_type.append(_sds((NC, N_PAD, DIM)))
        scratch += [
            pltpu.VMEM((NCH, K), jnp.int32),
            pltpu.VMEM((NCH, K), jnp.int32),
            pltpu.VMEM((NBUF, K, DIM), f32),
            pltpu.SemaphoreType.DMA((NBUF,)),
            pltpu.SemaphoreType.DMA((NBUF,)),
        ]
    if do_pool:
        out_type += [_sds((NW, BP * DIM)), _sds((NW, BP * DIM))]
        scratch += [
            pltpu.VMEM((NPT * DIM,), f32),
            pltpu.VMEM((BP,), jnp.int32),
            pltpu.VMEM((BP * DIM,), f32),
            pltpu.VMEM((BP * DIM,), f32),
        ]
    if do_agg:
        scratch.append(pltpu.VMEM_SHARED((N_PAD, DIM), f32))
    return pl.kernel(
        functools.partial(_agg_pool_body, do_agg, do_pool),
        out_type=tuple(out_type),
        mesh=_MESH,
        compiler_params=_SC_PARAMS,
        scratch_types=tuple(scratch),
    )


_sc_agg = _make_agg_pool(True, False)
_sc_agg_pool = _make_agg_pool(True, True)
_sc_pool = _make_agg_pool(False, True)


_RB = 1024
_GRID = N_PAD // _RB


def _t1_body(degp, x, w1, dinv, hn):
    deg = degp[0] + degp[1] + 1.0
    di = lax.rsqrt(deg)
    dinv[...] = di
    hn[...] = jnp.dot(x[...], w1[...], preferred_element_type=f32) * di


def _t1(degp, x_pad, W1):
    return pl.pallas_call(
        _t1_body,
        grid=(_GRID,),
        in_specs=[
            pl.BlockSpec((NC, _RB, 1), lambda i: (0, i, 0)),
            pl.BlockSpec((_RB, D), lambda i: (i, 0)),
            pl.BlockSpec((D, DIM), lambda i: (0, 0)),
        ],
        out_specs=[
            pl.BlockSpec((_RB, 1), lambda i: (i, 0)),
            pl.BlockSpec((_RB, DIM), lambda i: (i, 0)),
        ],
        out_shape=[_sds((N_PAD, 1)), _sds((N_PAD, DIM))],
    )(degp, x_pad, W1)


def _tmid_body(aggp, hn, dinv, b, w, hout, hnextn):
    di = dinv[...]
    h = jnp.maximum(di * (aggp[0] + aggp[1] + hn[...]) + b[...], 0.0)
    hout[...] = h
    hnextn[...] = jnp.dot(h, w[...], preferred_element_type=f32) * di


def _tmid(aggp, hn, dinv, b, Wnext):
    return pl.pallas_call(
        _tmid_body,
        grid=(_GRID,),
        in_specs=[
            pl.BlockSpec((NC, _RB, DIM), lambda i: (0, i, 0)),
            pl.BlockSpec((_RB, DIM), lambda i: (i, 0)),
            pl.BlockSpec((_RB, 1), lambda i: (i, 0)),
            pl.BlockSpec((1, DIM), lambda i: (0, 0)),
            pl.BlockSpec((DIM, DIM), lambda i: (0, 0)),
        ],
        out_specs=[
            pl.BlockSpec((_RB, DIM), lambda i: (i, 0)),
            pl.BlockSpec((_RB, DIM), lambda i: (i, 0)),
        ],
        out_shape=[_sds((N_PAD, DIM)), _sds((N_PAD, DIM))],
    )(aggp, hn, dinv, b, Wnext)


def _tlast_body(aggp, hn, dinv, b, hout):
    h = jnp.maximum(dinv[...] * (aggp[0] + aggp[1] + hn[...]) + b[...], 0.0)
    hout[...] = h


def _tlast(aggp, hn, dinv, b):
    return pl.pallas_call(
        _tlast_body,
        grid=(_GRID,),
        in_specs=[
            pl.BlockSpec((NC, _RB, DIM), lambda i: (0, i, 0)),
            pl.BlockSpec((_RB, DIM), lambda i: (i, 0)),
            pl.BlockSpec((_RB, 1), lambda i: (i, 0)),
            pl.BlockSpec((1, DIM), lambda i: (0, 0)),
        ],
        out_specs=pl.BlockSpec((_RB, DIM), lambda i: (i, 0)),
        out_shape=_sds((N_PAD, DIM)),
    )(aggp, hn, dinv, b)


def _head_body(ps1, pm1, ps2, pm2, ps3, pm3, cntp, wl1, bl1, wl2, bl2, wl3, bl3, out):
    cnt = jnp.maximum(cntp[0] + cntp[1], 1.0)

    def pooled(ps, pm):
        sm = jnp.sum(ps[...], axis=0)
        mx = jnp.max(pm[...], axis=0)
        mx = jnp.where(jnp.isfinite(mx), mx, 0.0)
        xl = jnp.concatenate([mx, sm / cnt], axis=1)
        return jnp.maximum(xl, 0.0)

    g = pooled(ps1, pm1) + pooled(ps2, pm2) + pooled(ps3, pm3)
    g = jnp.maximum(jnp.dot(g, wl1[...], preferred_element_type=f32) + bl1[...], 0.0)
    g = jnp.maximum(jnp.dot(g, wl2[...], preferred_element_type=f32) + bl2[...], 0.0)
    lg = jnp.dot(g, wl3[...], preferred_element_type=f32) + bl3[...]
    m = jnp.max(lg, axis=-1, keepdims=True)
    lse = jnp.log(jnp.sum(jnp.exp(lg - m), axis=-1, keepdims=True))
    out[...] = (lg - m - lse)[:B]


def _head(ps1, pm1, ps2, pm2, ps3, pm3, cntp, Wl1, bl1, Wl2, bl2, Wl3, bl3):
    return pl.pallas_call(_head_body, out_shape=_sds((B, C)))(
        ps1, pm1, ps2, pm2, ps3, pm3, cntp, Wl1, bl1, Wl2, bl2, Wl3, bl3
    )


def kernel(x, edge_index, batch, W1, b1, W2, b2, W3, b3, Wl1, bl1, Wl2, bl2, Wl3, bl3):
    src = edge_index[0]
    dst = edge_index[1]
    pad_e = NCH * K - EPT
    srcp = jnp.pad(
        src.reshape(NW, EPT), ((0, 0), (0, pad_e)), constant_values=0
    ).reshape(NW, NCH, K)
    dstp = jnp.pad(
        dst.reshape(NW, EPT), ((0, 0), (0, pad_e)), constant_values=TRASH
    ).reshape(NW, NCH, K)
    batp = jnp.pad(
        jnp.pad(batch, (0, N_PAD - N), constant_values=B).reshape(NW, NPT),
        ((0, 0), (0, BP - NPT)),
        constant_values=B,
    )
    batc = jnp.pad(batch, (0, NW * BCH * K - N), constant_values=B).reshape(
        NW, BCH, K
    )
    x_pad = jnp.pad(x, ((0, N_PAD - N), (0, 0)))

    zn = jnp.zeros((N_PAD // NS,), f32)
    z640 = jnp.zeros((N_PAD // NS, DIM), f32)
    zf = jnp.zeros((BP * DIM,), f32)
    negf = jnp.full((BP * DIM,), -jnp.inf, f32)

    degp, cntp = _sc_degcnt(dstp, batc, zn)

    dinv, h1n = _t1(degp.reshape(NC, N_PAD, 1), x_pad, W1)
    (aggp1,) = _sc_agg(h1n, srcp, dstp, z640)
    h1, h2n = _tmid(aggp1, h1n, dinv, b1.reshape(1, DIM), W2)

    aggp2, ps1, pm1 = _sc_agg_pool(
        h2n, srcp, dstp, z640, h1.reshape(-1), batp, zf, negf
    )
    h2, h3n = _tmid(aggp2, h2n, dinv, b2.reshape(1, DIM), W3)

    aggp3, ps2, pm2 = _sc_agg_pool(
        h3n, srcp, dstp, z640, h2.reshape(-1), batp, zf, negf
    )
    h3 = _tlast(aggp3, h3n, dinv, b3.reshape(1, DIM))

    ps3, pm3 = _sc_pool(h3.reshape(-1), batp, zf, negf)

    def prs(p):
        return p.reshape(NW, BP, DIM)

    return _head(
        prs(ps1), prs(pm1), prs(ps2), prs(pm2), prs(ps3), prs(pm3),
        cntp.reshape(NC, BP, 1),
        Wl1, bl1.reshape(1, DIM), Wl2, bl2.reshape(1, DIM // 2),
        Wl3, bl3.reshape(1, C),
    )


# speedup vs baseline: 1.3482x; 1.0457x over previous
import functools

import jax
import jax.numpy as jnp
from jax import lax
from jax.experimental import pallas as pl
from jax.experimental.pallas import tpu as pltpu
from jax.experimental.pallas import tpu_sc as plsc

N = 10000
E = 320000
D = 128
B = 256
DIM = 32
C = 10

NC = 2
NS = 16
NW = NC * NS

N_PAD = 10240
NPT = N_PAD // NW
EPT = E // NW
K = 128
NCH = (EPT + K - 1) // K
TRASH = N + 8
BP = 384
BCH = (BP + K - 1) // K
NBUF = 12
PD = 6

f32 = jnp.float32


def _sds(shape, dtype=f32):
    return jax.ShapeDtypeStruct(shape, dtype)


_MESH = plsc.VectorSubcoreMesh(
    core_axis_name="c", subcore_axis_name="s", num_cores=NC, num_subcores=NS
)


def _degcnt_body(dstp, batc, zn, deg_out, cnt_out, didx, bidx, ones_v, deg_sh, cnt_sh):
    c = lax.axis_index("c")
    s = lax.axis_index("s")
    wid = c * NS + s

    for i in range(K // 16):
        ones_v[pl.ds(i * 16, 16)] = jnp.ones((16,), f32)

    pltpu.sync_copy(zn, deg_sh.at[pl.ds(s * (N_PAD // NS), N_PAD // NS)])

    @pl.when(s == 0)
    def _():
        pltpu.sync_copy(zn.at[pl.ds(0, BP)], cnt_sh)

    pltpu.sync_copy(dstp.at[wid], didx)
    pltpu.sync_copy(batc.at[wid], bidx)
    plsc.subcore_barrier()

    def chunk(j, carry):
        pltpu.sync_copy(ones_v, deg_sh.at[didx.at[j]], add=True)
        return carry

    lax.fori_loop(0, NCH, chunk, 0)

    def bchunk(j, carry):
        pltpu.sync_copy(ones_v, cnt_sh.at[bidx.at[j]], add=True)
        return carry

    lax.fori_loop(0, BCH, bchunk, 0)

    plsc.subcore_barrier()
    pltpu.sync_copy(
        deg_sh.at[pl.ds(s * (N_PAD // NS), N_PAD // NS)],
        deg_out.at[c, pl.ds(s * (N_PAD // NS), N_PAD // NS)],
    )

    @pl.when(s == 0)
    def _():
        pltpu.sync_copy(cnt_sh, cnt_out.at[c])


_SC_PARAMS = pltpu.CompilerParams(
    use_tc_tiling_on_sc=False, needs_layout_passes=False
)

_sc_degcnt = pl.kernel(
    _degcnt_body,
    out_type=(_sds((NC, N_PAD)), _sds((NC, BP))),
    mesh=_MESH,
    compiler_params=_SC_PARAMS,
    scratch_types=(
        pltpu.VMEM((NCH, K), jnp.int32),
        pltpu.VMEM((BCH, K), jnp.int32),
        pltpu.VMEM((K,), f32),
        pltpu.VMEM_SHARED((N_PAD,), f32),
        pltpu.VMEM_SHARED((BP,), f32),
    ),
)


def _run_pool_loop(bbuf, psum, pmax, rowfn):
    lanes = lax.iota(jnp.int32, 16)

    def pnode(i, carry):
        bi = plsc.load_gather(bbuf, [jnp.full((16,), i, jnp.int32)])
        r0, r1 = rowfn(i, lanes)
        i0 = bi * DIM + lanes
        i1 = i0 + 16
        plsc.addupdate_scatter(psum, [i0], r0)
        plsc.addupdate_scatter(psum, [i1], r1)
        plsc.store_scatter(pmax, [i0], jnp.maximum(plsc.load_gather(pmax, [i0]), r0))
        plsc.store_scatter(pmax, [i1], jnp.maximum(plsc.load_gather(pmax, [i1]), r1))
        return carry

    lax.fori_loop(0, NPT, pnode, 0)


def _agg_pool_body(do_agg, do_pool, *refs):
    refs = list(refs)
    if do_agg:
        hn, srcp, dstp, z640 = refs[:4]
        del refs[:4]
    if do_pool:
        hprev_f, batp, zf, negf = refs[:4]
        del refs[:4]
    if do_agg:
        aggp = refs.pop(0)
    if do_pool:
        psump, pmaxp = refs[:2]
        del refs[:2]
    if do_agg:
        sidx, didx, rows, gsem, ssem = refs[:5]
        del refs[:5]
    if do_pool:
        hbuf, bbuf, psum, pmax = refs[:4]
        del refs[:4]
    if do_agg:
        agg_sh = refs.pop(0)

    c = lax.axis_index("c")
    s = lax.axis_index("s")
    wid = c * NS + s

    if do_agg:
        pltpu.sync_copy(z640, agg_sh.at[pl.ds(s * (N_PAD // NS), N_PAD // NS)])
        pltpu.sync_copy(srcp.at[wid], sidx)
        pltpu.sync_copy(dstp.at[wid], didx)
    if do_pool:
        pltpu.sync_copy(hprev_f.at[pl.ds(wid * NPT * DIM, NPT * DIM)], hbuf)
        pltpu.sync_copy(batp.at[wid], bbuf)
        pltpu.sync_copy(zf, psum)
        pltpu.sync_copy(negf, pmax)
    if do_agg:
        plsc.subcore_barrier()

        for b in range(PD):
            pltpu.async_copy(hn.at[sidx.at[b]], rows.at[b], gsem.at[b])

        def chunk(j, carry):
            slot = j % NBUF
            nxt = j + PD

            @pl.when(jnp.logical_and(nxt < NCH, j >= PD))
            def _():
                pltpu.make_async_copy(
                    rows.at[nxt % NBUF], agg_sh.at[didx.at[j]], ssem.at[nxt % NBUF]
                ).wait()

            @pl.when(nxt < NCH)
            def _():
                pltpu.async_copy(
                    hn.at[sidx.at[nxt]], rows.at[nxt % NBUF], gsem.at[nxt % NBUF]
                )

            pltpu.make_async_copy(
                hn.at[sidx.at[j]], rows.at[slot], gsem.at[slot]
            ).wait()
            pltpu.async_copy(
                rows.at[slot], agg_sh.at[didx.at[j]], ssem.at[slot], add=True
            )
            return carry

        lax.fori_loop(0, NCH, chunk, 0)
        for t in range(max(0, NCH - 2 * PD), NCH):
            pltpu.make_async_copy(
                rows.at[t % NBUF], agg_sh.at[didx.at[t]], ssem.at[t % NBUF]
            ).wait()

    if do_pool:
        _run_pool_loop(
            bbuf,
            psum,
            pmax,
            lambda i, lanes: (
                plsc.load_gather(hbuf, [i * DIM + lanes]),
                plsc.load_gather(hbuf, [i * DIM + 16 + lanes]),
            ),
        )

    if do_agg:
        plsc.subcore_barrier()
        pltpu.sync_copy(
            agg_sh.at[pl.ds(s * (N_PAD // NS), N_PAD // NS)],
            aggp.at[c, pl.ds(s * (N_PAD // NS), N_PAD // NS)],
        )
    if do_pool:
        pltpu.sync_copy(psum, psump.at[wid])
        pltpu.sync_copy(pmax, pmaxp.at[wid])


def _make_agg_pool(do_agg, do_pool):
    out_type = []
    scratch = []
    if do_agg:
        out_type.append(_sds((NC, N_PAD, DIM)))
        scratch += [
            pltpu.VMEM((NCH, K), jnp.int32),
            pltpu.VMEM((NCH, K), jnp.int32),
            pltpu.VMEM((NBUF, K, DIM), f32),
            pltpu.SemaphoreType.DMA((NBUF,)),
            pltpu.SemaphoreType.DMA((NBUF,)),
        ]
    if do_pool:
        out_type += [_sds((NW, BP * DIM)), _sds((NW, BP * DIM))]
        scratch += [
            pltpu.VMEM((NPT * DIM,), f32),
            pltpu.VMEM((BP,), jnp.int32),
            pltpu.VMEM((BP * DIM,), f32),
            pltpu.VMEM((BP * DIM,), f32),
        ]
    if do_agg:
        scratch.append(pltpu.VMEM_SHARED((N_PAD, DIM), f32))
    return pl.kernel(
        functools.partial(_agg_pool_body, do_agg, do_pool),
        out_type=tuple(out_type),
        mesh=_MESH,
        compiler_params=_SC_PARAMS,
        scratch_types=tuple(scratch),
    )


_sc_agg = _make_agg_pool(True, False)
_sc_agg_pool = _make_agg_pool(True, True)


def _pool3_body(
    aggf, hnf, dinv2, b3p, batp, zf, negf, psump, pmaxp,
    a0b, a1b, hnb, dib, b3b, bbuf, psum, pmax,
):
    c = lax.axis_index("c")
    s = lax.axis_index("s")
    wid = c * NS + s
    off = wid * NPT * DIM

    pltpu.sync_copy(aggf.at[0, pl.ds(off, NPT * DIM)], a0b)
    pltpu.sync_copy(aggf.at[1, pl.ds(off, NPT * DIM)], a1b)
    pltpu.sync_copy(hnf.at[pl.ds(off, NPT * DIM)], hnb)
    pltpu.sync_copy(dinv2.at[wid], dib)
    pltpu.sync_copy(b3p, b3b)
    pltpu.sync_copy(batp.at[wid], bbuf)
    pltpu.sync_copy(zf, psum)
    pltpu.sync_copy(negf, pmax)

    l16 = lax.iota(jnp.int32, 16)
    b3lo = plsc.load_gather(b3b, [l16])
    b3hi = plsc.load_gather(b3b, [l16 + 16])

    def rowfn(i, lanes):
        rep = jnp.full((16,), i, jnp.int32)
        di = plsc.load_gather(dib, [rep])
        lo = i * DIM + lanes
        hi = lo + 16
        r0 = di * (
            plsc.load_gather(a0b, [lo])
            + plsc.load_gather(a1b, [lo])
            + plsc.load_gather(hnb, [lo])
        ) + b3lo
        r1 = di * (
            plsc.load_gather(a0b, [hi])
            + plsc.load_gather(a1b, [hi])
            + plsc.load_gather(hnb, [hi])
        ) + b3hi
        return jnp.maximum(r0, 0.0), jnp.maximum(r1, 0.0)

    _run_pool_loop(bbuf, psum, pmax, rowfn)
    pltpu.sync_copy(psum, psump.at[wid])
    pltpu.sync_copy(pmax, pmaxp.at[wid])


_sc_pool3 = pl.kernel(
    _pool3_body,
    out_type=(_sds((NW, BP * DIM)), _sds((NW, BP * DIM))),
    mesh=_MESH,
    compiler_params=_SC_PARAMS,
    scratch_types=(
        pltpu.VMEM((NPT * DIM,), f32),
        pltpu.VMEM((NPT * DIM,), f32),
        pltpu.VMEM((NPT * DIM,), f32),
        pltpu.VMEM((BP,), f32),
        pltpu.VMEM((K,), f32),
        pltpu.VMEM((BP,), jnp.int32),
        pltpu.VMEM((BP * DIM,), f32),
        pltpu.VMEM((BP * DIM,), f32),
    ),
)


_RB = 1024
_GRID = N_PAD // _RB


def _t1_body(degp, x, w1, dinv, hn):
    deg = degp[0] + degp[1] + 1.0
    di = lax.rsqrt(deg)
    dinv[...] = di
    hn[...] = jnp.dot(x[...], w1[...], preferred_element_type=f32) * di


def _t1(degp, x_pad, W1):
    return pl.pallas_call(
        _t1_body,
        grid=(_GRID,),
        in_specs=[
            pl.BlockSpec((NC, _RB, 1), lambda i: (0, i, 0)),
            pl.BlockSpec((_RB, D), lambda i: (i, 0)),
            pl.BlockSpec((D, DIM), lambda i: (0, 0)),
        ],
        out_specs=[
            pl.BlockSpec((_RB, 1), lambda i: (i, 0)),
            pl.BlockSpec((_RB, DIM), lambda i: (i, 0)),
        ],
        out_shape=[_sds((N_PAD, 1)), _sds((N_PAD, DIM))],
    )(degp, x_pad, W1)


def _tmid_body(aggp, hn, dinv, b, w, hout, hnextn):
    di = dinv[...]
    h = jnp.maximum(di * (aggp[0] + aggp[1] + hn[...]) + b[...], 0.0)
    hout[...] = h
    hnextn[...] = jnp.dot(h, w[...], preferred_element_type=f32) * di


def _tmid(aggp, hn, dinv, b, Wnext):
    return pl.pallas_call(
        _tmid_body,
        grid=(_GRID,),
        in_specs=[
            pl.BlockSpec((NC, _RB, DIM), lambda i: (0, i, 0)),
            pl.BlockSpec((_RB, DIM), lambda i: (i, 0)),
            pl.BlockSpec((_RB, 1), lambda i: (i, 0)),
            pl.BlockSpec((1, DIM), lambda i: (0, 0)),
            pl.BlockSpec((DIM, DIM), lambda i: (0, 0)),
        ],
        out_specs=[
            pl.BlockSpec((_RB, DIM), lambda i: (i, 0)),
            pl.BlockSpec((_RB, DIM), lambda i: (i, 0)),
        ],
        out_shape=[_sds((N_PAD, DIM)), _sds((N_PAD, DIM))],
    )(aggp, hn, dinv, b, Wnext)


def _head_body(ps1, pm1, ps2, pm2, ps3, pm3, cntp, wl1, bl1, wl2, bl2, wl3, bl3, out):
    cnt = jnp.maximum(cntp[0] + cntp[1], 1.0)

    def pooled(ps, pm):
        sm = jnp.sum(ps[...], axis=0)
        mx = jnp.max(pm[...], axis=0)
        mx = jnp.where(jnp.isfinite(mx), mx, 0.0)
        xl = jnp.concatenate([mx, sm / cnt], axis=1)
        return jnp.maximum(xl, 0.0)

    g = pooled(ps1, pm1) + pooled(ps2, pm2) + pooled(ps3, pm3)
    g = jnp.maximum(jnp.dot(g, wl1[...], preferred_element_type=f32) + bl1[...], 0.0)
    g = jnp.maximum(jnp.dot(g, wl2[...], preferred_element_type=f32) + bl2[...], 0.0)
    lg = jnp.dot(g, wl3[...], preferred_element_type=f32) + bl3[...]
    m = jnp.max(lg, axis=-1, keepdims=True)
    lse = jnp.log(jnp.sum(jnp.exp(lg - m), axis=-1, keepdims=True))
    out[...] = (lg - m - lse)[:B]


def _head(ps1, pm1, ps2, pm2, ps3, pm3, cntp, Wl1, bl1, Wl2, bl2, Wl3, bl3):
    return pl.pallas_call(_head_body, out_shape=_sds((B, C)))(
        ps1, pm1, ps2, pm2, ps3, pm3, cntp, Wl1, bl1, Wl2, bl2, Wl3, bl3
    )


def kernel(x, edge_index, batch, W1, b1, W2, b2, W3, b3, Wl1, bl1, Wl2, bl2, Wl3, bl3):
    src = edge_index[0]
    dst = edge_index[1]
    pad_e = NCH * K - EPT
    srcp = jnp.pad(
        src.reshape(NW, EPT), ((0, 0), (0, pad_e)), constant_values=0
    ).reshape(NW, NCH, K)
    dstp = jnp.pad(
        dst.reshape(NW, EPT), ((0, 0), (0, pad_e)), constant_values=TRASH
    ).reshape(NW, NCH, K)
    batp = jnp.pad(
        jnp.pad(batch, (0, N_PAD - N), constant_values=B).reshape(NW, NPT),
        ((0, 0), (0, BP - NPT)),
        constant_values=B,
    )
    batc = jnp.pad(batch, (0, NW * BCH * K - N), constant_values=B).reshape(
        NW, BCH, K
    )
    x_pad = jnp.pad(x, ((0, N_PAD - N), (0, 0)))

    zn = jnp.zeros((N_PAD // NS,), f32)
    z640 = jnp.zeros((N_PAD // NS, DIM), f32)
    zf = jnp.zeros((BP * DIM,), f32)
    negf = jnp.full((BP * DIM,), -jnp.inf, f32)

    degp, cntp = _sc_degcnt(dstp, batc, zn)

    dinv, h1n = _t1(degp.reshape(NC, N_PAD, 1), x_pad, W1)
    (aggp1,) = _sc_agg(h1n, srcp, dstp, z640)
    h1, h2n = _tmid(aggp1, h1n, dinv, b1.reshape(1, DIM), W2)

    aggp2, ps1, pm1 = _sc_agg_pool(
        h2n, srcp, dstp, z640, h1.reshape(-1), batp, zf, negf
    )
    h2, h3n = _tmid(aggp2, h2n, dinv, b2.reshape(1, DIM), W3)

    aggp3, ps2, pm2 = _sc_agg_pool(
        h3n, srcp, dstp, z640, h2.reshape(-1), batp, zf, negf
    )

    dinv2 = jnp.pad(dinv.reshape(NW, NPT), ((0, 0), (0, BP - NPT)))
    b3p = jnp.pad(b3, (0, K - DIM))
    ps3, pm3 = _sc_pool3(
        aggp3.reshape(NC, N_PAD * DIM), h3n.reshape(-1), dinv2, b3p, batp, zf, negf
    )

    def prs(p):
        return p.reshape(NW, BP, DIM)

    return _head(
        prs(ps1), prs(pm1), prs(ps2), prs(pm2), prs(ps3), prs(pm3),
        cntp.reshape(NC, BP, 1),
        Wl1, bl1.reshape(1, DIM), Wl2, bl2.reshape(1, DIM // 2),
        Wl3, bl3.reshape(1, C),
    )
